# baseline (speedup 1.0000x reference)
import functools

import jax
import jax.numpy as jnp
from jax import lax
from jax.experimental import pallas as pl
from jax.experimental.pallas import tpu as pltpu
from jax.experimental.pallas import tpu_sc as plsc

NC = 2
NS = 16
NW = NC * NS
L = 16

K = 32
D = 128
G = 2
CPW = 320
NP = NW * CPW
NG = CPW // G
NPAIR = NG // 2

_mesh = plsc.VectorSubcoreMesh(
    core_axis_name="c", subcore_axis_name="s", num_cores=NC, num_subcores=NS)

_f32 = jnp.float32
_i32 = jnp.int32


def _rsqrt_newton(sq):
    bits = plsc.bitcast(sq, _i32)
    y = plsc.bitcast(jnp.int32(0x5F3759DF) - lax.shift_right_logical(bits, 1),
                     _f32)
    for _ in range(3):
        y = y * (1.5 - 0.5 * sq * y * y)
    return y


@functools.partial(
    pl.kernel,
    out_type=(jax.ShapeDtypeStruct((NP * K,), _f32),
              jax.ShapeDtypeStruct((NP * K,), _i32)),
    mesh=_mesh,
    compiler_params=pltpu.CompilerParams(needs_layout_passes=False),
    scratch_types=[
        pltpu.VMEM((NP,), _f32),
        pltpu.VMEM((NP,), _f32),
        pltpu.VMEM((NP,), _f32),
        pltpu.VMEM((CPW, K), _i32),
        pltpu.VMEM((CPW * K,), _f32),
        pltpu.VMEM((CPW * K,), _i32),
    ],
)
def _sc_prep(px_hbm, py_hbm, pz_hbm, nbr_hbm, w_out, fidx_out,
             px_v, py_v, pz_v, nbr_v, w_v, fidx_v):
    wid = lax.axis_index("s") * NC + lax.axis_index("c")
    base = wid * CPW
    pltpu.sync_copy(px_hbm, px_v)
    pltpu.sync_copy(py_hbm, py_v)
    pltpu.sync_copy(pz_hbm, pz_v)
    pltpu.sync_copy(nbr_hbm.at[pl.ds(base, CPW)], nbr_v)
    zero16 = jnp.zeros((L,), _i32)
    lane = jnp.arange(L, dtype=_i32)

    def body(j, _):
        ja16 = zero16 + (base + j)
        px = plsc.load_gather(px_v, [ja16])
        py = plsc.load_gather(py_v, [ja16])
        pz = plsc.load_gather(pz_v, [ja16])
        for hk in range(K // L):
            nb = nbr_v[j, pl.ds(hk * L, L)]
            gx = plsc.load_gather(px_v, [nb])
            gy = plsc.load_gather(py_v, [nb])
            gz = plsc.load_gather(pz_v, [nb])
            dx = px - gx
            dy = py - gy
            dz = pz - gz
            sq = dx * dx + dy * dy + dz * dz
            w16 = jnp.where(sq == 0.0, jnp.float32(2.0), _rsqrt_newton(sq))
            fi = nb + (lane + hk * L) * NP
            off = j * K + hk * L
            w_v[pl.ds(off, L)] = w16
            fidx_v[pl.ds(off, L)] = fi
        return 0

    lax.fori_loop(0, CPW, body, 0)
    pltpu.sync_copy(w_v, w_out.at[pl.ds(base * K, CPW * K)])
    pltpu.sync_copy(fidx_v, fidx_out.at[pl.ds(base * K, CPW * K)])


def _make_combine(act):
    @functools.partial(
        pl.kernel,
        out_type=jax.ShapeDtypeStruct((NP, D), _f32),
        mesh=_mesh,
        compiler_params=pltpu.CompilerParams(needs_layout_passes=False),
        scratch_types=[
            pltpu.VMEM((NG, G * K), _i32),
            pltpu.VMEM((NG, G * K), _f32),
            pltpu.VMEM((G * K, D), _f32),
            pltpu.VMEM((G * K, D), _f32),
            pltpu.VMEM((G * K, D), _f32),
            pltpu.VMEM((G * K, D), _f32),
            pltpu.VMEM((CPW, D), _f32),
            pltpu.VMEM((1, D), _f32),
            pltpu.SemaphoreType.DMA,
            pltpu.SemaphoreType.DMA,
            pltpu.SemaphoreType.DMA,
            pltpu.SemaphoreType.DMA,
        ],
    )
    def _combine(p_hbm, fidx_hbm, w_hbm, b_hbm, out_hbm,
                 fidx_v, w_v, r0, r1, r2, r3, out_v, b_v, s0, s1, s2, s3):
        wid = lax.axis_index("s") * NC + lax.axis_index("c")
        rows_bufs = (r0, r1, r2, r3)
        sems = (s0, s1, s2, s3)
        pltpu.sync_copy(fidx_hbm.at[wid], fidx_v)
        pltpu.sync_copy(w_hbm.at[wid], w_v)
        pltpu.sync_copy(b_hbm, b_v)
        bch = [b_v[0, pl.ds(c * L, L)] for c in range(D // L)]

        def gather(g, slot):
            return pltpu.make_async_copy(p_hbm.at[fidx_v.at[g]],
                                         rows_bufs[slot], sems[slot])

        def compute(g, slot):
            rows = rows_bufs[slot]
            for jj in range(G):
                acc = list(bch)
                wrow = [w_v[g, pl.ds(jj * K + hk * L, L)]
                        for hk in range(K // L)]
                for r in range(K):
                    ws = wrow[r // L][r % L]
                    for c in range(D // L):
                        acc[c] = acc[c] + ws * rows[jj * K + r,
                                                    pl.ds(c * L, L)]
                node = g * G + jj
                for c in range(D // L):
                    v = acc[c]
                    if act:
                        v = jnp.maximum(v, 0.01 * v)
                    out_v[node, pl.ds(c * L, L)] = v

        for g0 in range(3):
            gather(g0, g0).start()

        def body(i, _):
            for j in range(4):
                g = i * 4 + j
                gather(g, j).wait()

                @pl.when(g + 3 < NG)
                def _():
                    gather(g + 3, (j + 3) % 4).start()

                compute(g, j)
            return 0

        lax.fori_loop(0, NG // 4, body, 0)
        pltpu.sync_copy(out_v, out_hbm.at[pl.ds(wid * CPW, CPW)])

    return _combine


_combine_noact = _make_combine(False)
_combine_act = _make_combine(True)


_MM_BN = 512
_KW = 8


def _mm_body(h_ref, m_ref, o_ref):
    h_blk = h_ref[...]
    for kk in range(K // _KW):
        r = jnp.dot(h_blk, m_ref[:, kk * _KW * D:(kk + 1) * _KW * D],
                    preferred_element_type=_f32)
        for j in range(_KW):
            o_ref[kk * _KW + j, :, :] = r[:, j * D:(j + 1) * D]


def _tc_matmul(h, m):
    bn = _MM_BN
    return pl.pallas_call(
        _mm_body,
        grid=(NP // bn,),
        in_specs=[
            pl.BlockSpec((bn, D), lambda i: (i, 0)),
            pl.BlockSpec((D, K * D), lambda i: (0, 0)),
        ],
        out_specs=pl.BlockSpec((K, bn, D), lambda i: (0, i, 0)),
        out_shape=jax.ShapeDtypeStruct((K, NP, D), _f32),
    )(h, m)


def kernel(x, pos, neighbors, W0, b0, W1, b1, W2, b2):
    n = x.shape[0]
    x_p = jnp.zeros((NP, D), _f32).at[:n].set(x)
    pos_p = jnp.zeros((NP, 3), _f32).at[:n].set(pos)
    nbr_p = jnp.zeros((NP, K), _i32).at[:n].set(neighbors)

    w_flat, fidx_flat = _sc_prep(pos_p[:, 0], pos_p[:, 1], pos_p[:, 2], nbr_p)
    w3 = w_flat.reshape(NW, NG, G * K)
    fidx3 = fidx_flat.reshape(NW, NG, G * K)

    h = x_p
    for wgt, b, act in ((W0, b0, False), (W1, b1, True), (W2, b2, False)):
        m = wgt.reshape(K, D, D).transpose(1, 0, 2).reshape(D, K * D)
        p = _tc_matmul(h, m).reshape(K * NP, D)
        comb = _combine_act if act else _combine_noact
        h = comb(p, fidx3, w3, b)
    return h[:n]


# speedup vs baseline: 1.2047x; 1.2047x over previous
import functools

import jax
import jax.numpy as jnp
from jax import lax
from jax.experimental import pallas as pl
from jax.experimental.pallas import tpu as pltpu
from jax.experimental.pallas import tpu_sc as plsc

NC = 2
NS = 16
NW = NC * NS
L = 16

K = 32
D = 128
G = 2
CPW = 320
NP = NW * CPW
NG = CPW // G
NPAIR = NG // 2

_mesh = plsc.VectorSubcoreMesh(
    core_axis_name="c", subcore_axis_name="s", num_cores=NC, num_subcores=NS)

_f32 = jnp.float32
_i32 = jnp.int32


def _rsqrt_newton(sq):
    bits = plsc.bitcast(sq, _i32)
    y = plsc.bitcast(jnp.int32(0x5F3759DF) - lax.shift_right_logical(bits, 1),
                     _f32)
    for _ in range(3):
        y = y * (1.5 - 0.5 * sq * y * y)
    return y


@functools.partial(
    pl.kernel,
    out_type=(jax.ShapeDtypeStruct((NP * K,), _f32),
              jax.ShapeDtypeStruct((NP * K,), _i32)),
    mesh=_mesh,
    compiler_params=pltpu.CompilerParams(needs_layout_passes=False),
    scratch_types=[
        pltpu.VMEM((NP,), _f32),
        pltpu.VMEM((NP,), _f32),
        pltpu.VMEM((NP,), _f32),
        pltpu.VMEM((CPW, K), _i32),
        pltpu.VMEM((CPW * K,), _f32),
        pltpu.VMEM((CPW * K,), _i32),
    ],
)
def _sc_prep(px_hbm, py_hbm, pz_hbm, nbr_hbm, w_out, fidx_out,
             px_v, py_v, pz_v, nbr_v, w_v, fidx_v):
    wid = lax.axis_index("s") * NC + lax.axis_index("c")
    base = wid * CPW
    pltpu.sync_copy(px_hbm, px_v)
    pltpu.sync_copy(py_hbm, py_v)
    pltpu.sync_copy(pz_hbm, pz_v)
    pltpu.sync_copy(nbr_hbm.at[pl.ds(base, CPW)], nbr_v)
    zero16 = jnp.zeros((L,), _i32)
    lane = jnp.arange(L, dtype=_i32)

    def body(j, _):
        ja16 = zero16 + (base + j)
        px = plsc.load_gather(px_v, [ja16])
        py = plsc.load_gather(py_v, [ja16])
        pz = plsc.load_gather(pz_v, [ja16])
        for hk in range(K // L):
            nb = nbr_v[j, pl.ds(hk * L, L)]
            gx = plsc.load_gather(px_v, [nb])
            gy = plsc.load_gather(py_v, [nb])
            gz = plsc.load_gather(pz_v, [nb])
            dx = px - gx
            dy = py - gy
            dz = pz - gz
            sq = dx * dx + dy * dy + dz * dz
            w16 = jnp.where(sq == 0.0, jnp.float32(2.0), _rsqrt_newton(sq))
            fi = nb + (lane + hk * L) * NP
            off = j * K + hk * L
            w_v[pl.ds(off, L)] = w16
            fidx_v[pl.ds(off, L)] = fi
        return 0

    lax.fori_loop(0, CPW, body, 0)
    pltpu.sync_copy(w_v, w_out.at[pl.ds(base * K, CPW * K)])
    pltpu.sync_copy(fidx_v, fidx_out.at[pl.ds(base * K, CPW * K)])


def _make_combine(act):
    @functools.partial(
        pl.kernel,
        out_type=jax.ShapeDtypeStruct((NP, D), _f32),
        mesh=_mesh,
        compiler_params=pltpu.CompilerParams(needs_layout_passes=False),
        scratch_types=[
            pltpu.VMEM((NG, G * K), _i32),
            pltpu.VMEM((NG * G * K,), _f32),
            pltpu.VMEM((G * K, D), _f32),
            pltpu.VMEM((G * K, D), _f32),
            pltpu.VMEM((CPW, D), _f32),
            pltpu.VMEM((1, D), _f32),
            pltpu.SemaphoreType.DMA,
            pltpu.SemaphoreType.DMA,
        ],
    )
    def _combine(p_hbm, fidx_hbm, w_hbm, b_hbm, out_hbm,
                 fidx_v, w_v, rows0, rows1, out_v, b_v, s0, s1):
        wid = lax.axis_index("s") * NC + lax.axis_index("c")
        pltpu.sync_copy(fidx_hbm.at[wid], fidx_v)
        pltpu.sync_copy(w_hbm.at[wid], w_v)
        pltpu.sync_copy(b_hbm, b_v)
        bch = [b_v[0, pl.ds(c * L, L)] for c in range(D // L)]
        zero16 = jnp.zeros((L,), _i32)
        nch = D // L

        def gather(g, rows, sem):
            return pltpu.make_async_copy(p_hbm.at[fidx_v.at[g]], rows, sem)

        def compute(g, rows):
            wbase = g * (G * K)

            def rbody(r, accs):
                ws0 = plsc.load_gather(w_v, [zero16 + (wbase + r)])
                ws1 = plsc.load_gather(w_v, [zero16 + (wbase + K + r)])
                out = []
                for c in range(nch):
                    out.append(accs[c] + ws0 * rows[r, pl.ds(c * L, L)])
                for c in range(nch):
                    out.append(accs[nch + c]
                               + ws1 * rows[K + r, pl.ds(c * L, L)])
                return tuple(out)

            accs = lax.fori_loop(0, K, rbody, tuple(bch * G))
            for jj in range(G):
                node = g * G + jj
                for c in range(nch):
                    v = accs[jj * nch + c]
                    if act:
                        v = jnp.maximum(v, 0.01 * v)
                    out_v[node, pl.ds(c * L, L)] = v

        gather(0, rows0, s0).start()

        def body(p, _):
            g0 = p * 2
            g1 = g0 + 1
            gather(g1, rows1, s1).start()
            gather(g0, rows0, s0).wait()
            compute(g0, rows0)

            @pl.when(p < NPAIR - 1)
            def _():
                gather(g0 + 2, rows0, s0).start()

            gather(g1, rows1, s1).wait()
            compute(g1, rows1)
            return 0

        lax.fori_loop(0, NPAIR, body, 0)
        pltpu.sync_copy(out_v, out_hbm.at[pl.ds(wid * CPW, CPW)])

    return _combine


_combine_noact = _make_combine(False)
_combine_act = _make_combine(True)


_MM_BN = 512
_KW = 8


def _mm_body(h_ref, m_ref, o_ref):
    h_blk = h_ref[...]
    for kk in range(K // _KW):
        r = jnp.dot(h_blk, m_ref[:, kk * _KW * D:(kk + 1) * _KW * D],
                    preferred_element_type=_f32)
        for j in range(_KW):
            o_ref[kk * _KW + j, :, :] = r[:, j * D:(j + 1) * D]


def _tc_matmul(h, m):
    bn = _MM_BN
    return pl.pallas_call(
        _mm_body,
        grid=(NP // bn,),
        in_specs=[
            pl.BlockSpec((bn, D), lambda i: (i, 0)),
            pl.BlockSpec((D, K * D), lambda i: (0, 0)),
        ],
        out_specs=pl.BlockSpec((K, bn, D), lambda i: (0, i, 0)),
        out_shape=jax.ShapeDtypeStruct((K, NP, D), _f32),
    )(h, m)


def kernel(x, pos, neighbors, W0, b0, W1, b1, W2, b2):
    n = x.shape[0]
    x_p = jnp.zeros((NP, D), _f32).at[:n].set(x)
    pos_p = jnp.zeros((NP, 3), _f32).at[:n].set(pos)
    nbr_p = jnp.zeros((NP, K), _i32).at[:n].set(neighbors)

    w_flat, fidx_flat = _sc_prep(pos_p[:, 0], pos_p[:, 1], pos_p[:, 2], nbr_p)
    w3 = w_flat.reshape(NW, NG * G * K)
    fidx3 = fidx_flat.reshape(NW, NG, G * K)

    h = x_p
    for wgt, b, act in ((W0, b0, False), (W1, b1, True), (W2, b2, False)):
        m = wgt.reshape(K, D, D).transpose(1, 0, 2).reshape(D, K * D)
        p = _tc_matmul(h, m).reshape(K * NP, D)
        comb = _combine_act if act else _combine_noact
        h = comb(p, fidx3, w3, b)
    return h[:n]


# speedup vs baseline: 1.3785x; 1.1443x over previous
import functools

import jax
import jax.numpy as jnp
from jax import lax
from jax.experimental import pallas as pl
from jax.experimental.pallas import tpu as pltpu
from jax.experimental.pallas import tpu_sc as plsc

NC = 2
NS = 16
NW = NC * NS
L = 16

K = 32
D = 128
G = 4
CPW = 320
NP = NW * CPW
NG = CPW // G
NPAIR = NG // 2

_mesh = plsc.VectorSubcoreMesh(
    core_axis_name="c", subcore_axis_name="s", num_cores=NC, num_subcores=NS)

_f32 = jnp.float32
_i32 = jnp.int32


def _rsqrt_newton(sq):
    bits = plsc.bitcast(sq, _i32)
    y = plsc.bitcast(jnp.int32(0x5F3759DF) - lax.shift_right_logical(bits, 1),
                     _f32)
    for _ in range(3):
        y = y * (1.5 - 0.5 * sq * y * y)
    return y


@functools.partial(
    pl.kernel,
    out_type=(jax.ShapeDtypeStruct((NP * K,), _f32),
              jax.ShapeDtypeStruct((NP * K,), _i32)),
    mesh=_mesh,
    compiler_params=pltpu.CompilerParams(needs_layout_passes=False),
    scratch_types=[
        pltpu.VMEM((NP,), _f32),
        pltpu.VMEM((NP,), _f32),
        pltpu.VMEM((NP,), _f32),
        pltpu.VMEM((CPW, K), _i32),
        pltpu.VMEM((CPW * K,), _f32),
        pltpu.VMEM((CPW * K,), _i32),
    ],
)
def _sc_prep(px_hbm, py_hbm, pz_hbm, nbr_hbm, w_out, fidx_out,
             px_v, py_v, pz_v, nbr_v, w_v, fidx_v):
    wid = lax.axis_index("s") * NC + lax.axis_index("c")
    base = wid * CPW
    pltpu.sync_copy(px_hbm, px_v)
    pltpu.sync_copy(py_hbm, py_v)
    pltpu.sync_copy(pz_hbm, pz_v)
    pltpu.sync_copy(nbr_hbm.at[pl.ds(base, CPW)], nbr_v)
    zero16 = jnp.zeros((L,), _i32)
    lane = jnp.arange(L, dtype=_i32)

    def body(j, _):
        ja16 = zero16 + (base + j)
        px = plsc.load_gather(px_v, [ja16])
        py = plsc.load_gather(py_v, [ja16])
        pz = plsc.load_gather(pz_v, [ja16])
        for hk in range(K // L):
            nb = nbr_v[j, pl.ds(hk * L, L)]
            gx = plsc.load_gather(px_v, [nb])
            gy = plsc.load_gather(py_v, [nb])
            gz = plsc.load_gather(pz_v, [nb])
            dx = px - gx
            dy = py - gy
            dz = pz - gz
            sq = dx * dx + dy * dy + dz * dz
            w16 = jnp.where(sq == 0.0, jnp.float32(2.0), _rsqrt_newton(sq))
            fi = nb + (lane + hk * L) * NP
            off = j * K + hk * L
            w_v[pl.ds(off, L)] = w16
            fidx_v[pl.ds(off, L)] = fi
        return 0

    lax.fori_loop(0, CPW, body, 0)
    pltpu.sync_copy(w_v, w_out.at[pl.ds(base * K, CPW * K)])
    pltpu.sync_copy(fidx_v, fidx_out.at[pl.ds(base * K, CPW * K)])


def _make_combine(act):
    @functools.partial(
        pl.kernel,
        out_type=jax.ShapeDtypeStruct((NP, D), _f32),
        mesh=_mesh,
        compiler_params=pltpu.CompilerParams(needs_layout_passes=False),
        scratch_types=[
            pltpu.VMEM((NG, G * K), _i32),
            pltpu.VMEM((NG * G * K,), _f32),
            pltpu.VMEM((G * K, D), _f32),
            pltpu.VMEM((G * K, D), _f32),
            pltpu.VMEM((CPW, D), _f32),
            pltpu.VMEM((1, D), _f32),
            pltpu.SemaphoreType.DMA,
            pltpu.SemaphoreType.DMA,
        ],
    )
    def _combine(p_hbm, fidx_hbm, w_hbm, b_hbm, out_hbm,
                 fidx_v, w_v, rows0, rows1, out_v, b_v, s0, s1):
        wid = lax.axis_index("s") * NC + lax.axis_index("c")
        pltpu.sync_copy(fidx_hbm.at[wid], fidx_v)
        pltpu.sync_copy(w_hbm.at[wid], w_v)
        pltpu.sync_copy(b_hbm, b_v)
        bch = [b_v[0, pl.ds(c * L, L)] for c in range(D // L)]
        zero16 = jnp.zeros((L,), _i32)
        nch = D // L

        def gather(g, rows, sem):
            return pltpu.make_async_copy(p_hbm.at[fidx_v.at[g]], rows, sem)

        def compute(g, rows):
            wbase = g * (G * K)

            def rbody(r, accs):
                out = []
                for jj in range(G):
                    ws = plsc.load_gather(
                        w_v, [zero16 + (wbase + jj * K + r)])
                    for c in range(nch):
                        out.append(accs[jj * nch + c]
                                   + ws * rows[jj * K + r, pl.ds(c * L, L)])
                return tuple(out)

            accs = lax.fori_loop(0, K, rbody, tuple(bch * G))
            for jj in range(G):
                node = g * G + jj
                for c in range(nch):
                    v = accs[jj * nch + c]
                    if act:
                        v = jnp.maximum(v, 0.01 * v)
                    out_v[node, pl.ds(c * L, L)] = v

        gather(0, rows0, s0).start()

        def body(p, _):
            g0 = p * 2
            g1 = g0 + 1
            gather(g1, rows1, s1).start()
            gather(g0, rows0, s0).wait()
            compute(g0, rows0)

            @pl.when(p < NPAIR - 1)
            def _():
                gather(g0 + 2, rows0, s0).start()

            gather(g1, rows1, s1).wait()
            compute(g1, rows1)
            return 0

        lax.fori_loop(0, NPAIR, body, 0)
        pltpu.sync_copy(out_v, out_hbm.at[pl.ds(wid * CPW, CPW)])

    return _combine


_combine_noact = _make_combine(False)
_combine_act = _make_combine(True)


_MM_BN = 512
_KW = 8


def _mm_body(h_ref, m_ref, o_ref):
    h_blk = h_ref[...]
    for kk in range(K // _KW):
        r = jnp.dot(h_blk, m_ref[:, kk * _KW * D:(kk + 1) * _KW * D],
                    preferred_element_type=_f32)
        for j in range(_KW):
            o_ref[kk * _KW + j, :, :] = r[:, j * D:(j + 1) * D]


def _tc_matmul(h, m):
    bn = _MM_BN
    return pl.pallas_call(
        _mm_body,
        grid=(NP // bn,),
        in_specs=[
            pl.BlockSpec((bn, D), lambda i: (i, 0)),
            pl.BlockSpec((D, K * D), lambda i: (0, 0)),
        ],
        out_specs=pl.BlockSpec((K, bn, D), lambda i: (0, i, 0)),
        out_shape=jax.ShapeDtypeStruct((K, NP, D), _f32),
    )(h, m)


def kernel(x, pos, neighbors, W0, b0, W1, b1, W2, b2):
    n = x.shape[0]
    x_p = jnp.zeros((NP, D), _f32).at[:n].set(x)
    pos_p = jnp.zeros((NP, 3), _f32).at[:n].set(pos)
    nbr_p = jnp.zeros((NP, K), _i32).at[:n].set(neighbors)

    w_flat, fidx_flat = _sc_prep(pos_p[:, 0], pos_p[:, 1], pos_p[:, 2], nbr_p)
    w3 = w_flat.reshape(NW, NG * G * K)
    fidx3 = fidx_flat.reshape(NW, NG, G * K)

    h = x_p
    for wgt, b, act in ((W0, b0, False), (W1, b1, True), (W2, b2, False)):
        m = wgt.reshape(K, D, D).transpose(1, 0, 2).reshape(D, K * D)
        p = _tc_matmul(h, m).reshape(K * NP, D)
        comb = _combine_act if act else _combine_noact
        h = comb(p, fidx3, w3, b)
    return h[:n]


# speedup vs baseline: 1.4527x; 1.0538x over previous
import functools

import jax
import jax.numpy as jnp
from jax import lax
from jax.experimental import pallas as pl
from jax.experimental.pallas import tpu as pltpu
from jax.experimental.pallas import tpu_sc as plsc

NC = 2
NS = 16
NW = NC * NS
L = 16

K = 32
D = 128
G = 4
CPW = 320
NP = NW * CPW
NG = CPW // G
NPAIR = NG // 2

_mesh = plsc.VectorSubcoreMesh(
    core_axis_name="c", subcore_axis_name="s", num_cores=NC, num_subcores=NS)

_f32 = jnp.float32
_i32 = jnp.int32


def _rsqrt_newton(sq):
    bits = plsc.bitcast(sq, _i32)
    y = plsc.bitcast(jnp.int32(0x5F3759DF) - lax.shift_right_logical(bits, 1),
                     _f32)
    for _ in range(3):
        y = y * (1.5 - 0.5 * sq * y * y)
    return y


@functools.partial(
    pl.kernel,
    out_type=(jax.ShapeDtypeStruct((NP * K,), _f32),
              jax.ShapeDtypeStruct((NP * K,), _i32)),
    mesh=_mesh,
    compiler_params=pltpu.CompilerParams(needs_layout_passes=False),
    scratch_types=[
        pltpu.VMEM((NP,), _f32),
        pltpu.VMEM((NP,), _f32),
        pltpu.VMEM((NP,), _f32),
        pltpu.VMEM((CPW, K), _i32),
        pltpu.VMEM((CPW * K,), _f32),
        pltpu.VMEM((CPW * K,), _i32),
    ],
)
def _sc_prep(px_hbm, py_hbm, pz_hbm, nbr_hbm, w_out, fidx_out,
             px_v, py_v, pz_v, nbr_v, w_v, fidx_v):
    wid = lax.axis_index("s") * NC + lax.axis_index("c")
    base = wid * CPW
    pltpu.sync_copy(px_hbm, px_v)
    pltpu.sync_copy(py_hbm, py_v)
    pltpu.sync_copy(pz_hbm, pz_v)
    pltpu.sync_copy(nbr_hbm.at[pl.ds(base, CPW)], nbr_v)
    zero16 = jnp.zeros((L,), _i32)
    lane = jnp.arange(L, dtype=_i32)

    def body(j, _):
        ja16 = zero16 + (base + j)
        px = plsc.load_gather(px_v, [ja16])
        py = plsc.load_gather(py_v, [ja16])
        pz = plsc.load_gather(pz_v, [ja16])
        for hk in range(K // L):
            nb = nbr_v[j, pl.ds(hk * L, L)]
            gx = plsc.load_gather(px_v, [nb])
            gy = plsc.load_gather(py_v, [nb])
            gz = plsc.load_gather(pz_v, [nb])
            dx = px - gx
            dy = py - gy
            dz = pz - gz
            sq = dx * dx + dy * dy + dz * dz
            w16 = jnp.where(sq == 0.0, jnp.float32(2.0), _rsqrt_newton(sq))
            fi = nb + (lane + hk * L) * NP
            off = j * K + hk * L
            w_v[pl.ds(off, L)] = w16
            fidx_v[pl.ds(off, L)] = fi
        return 0

    lax.fori_loop(0, CPW, body, 0)
    pltpu.sync_copy(w_v, w_out.at[pl.ds(base * K, CPW * K)])
    pltpu.sync_copy(fidx_v, fidx_out.at[pl.ds(base * K, CPW * K)])


def _make_combine(act):
    @functools.partial(
        pl.kernel,
        out_type=jax.ShapeDtypeStruct((NP, D), _f32),
        mesh=_mesh,
        compiler_params=pltpu.CompilerParams(needs_layout_passes=False),
        scratch_types=[
            pltpu.VMEM((NG, G * K), _i32),
            pltpu.VMEM((NG * G * K,), _f32),
            pltpu.VMEM((G * K, D), _f32),
            pltpu.VMEM((G * K, D), _f32),
            pltpu.VMEM((G * K, D), _f32),
            pltpu.VMEM((G * K, D), _f32),
            pltpu.VMEM((CPW, D), _f32),
            pltpu.VMEM((1, D), _f32),
            pltpu.SemaphoreType.DMA,
            pltpu.SemaphoreType.DMA,
            pltpu.SemaphoreType.DMA,
            pltpu.SemaphoreType.DMA,
        ],
    )
    def _combine(p_hbm, fidx_hbm, w_hbm, b_hbm, out_hbm,
                 fidx_v, w_v, rows0, rows1, rows2, rows3, out_v, b_v,
                 s0, s1, s2, s3):
        wid = lax.axis_index("s") * NC + lax.axis_index("c")
        rows_bufs = (rows0, rows1, rows2, rows3)
        sems = (s0, s1, s2, s3)
        pltpu.sync_copy(fidx_hbm.at[wid], fidx_v)
        pltpu.sync_copy(w_hbm.at[wid], w_v)
        pltpu.sync_copy(b_hbm, b_v)
        bch = [b_v[0, pl.ds(c * L, L)] for c in range(D // L)]
        zero16 = jnp.zeros((L,), _i32)
        nch = D // L

        def gather(g, rows, sem):
            return pltpu.make_async_copy(p_hbm.at[fidx_v.at[g]], rows, sem)

        def compute(g, rows):
            wbase = g * (G * K)

            def rbody(r, accs):
                out = []
                for jj in range(G):
                    ws = plsc.load_gather(
                        w_v, [zero16 + (wbase + jj * K + r)])
                    for c in range(nch):
                        out.append(accs[jj * nch + c]
                                   + ws * rows[jj * K + r, pl.ds(c * L, L)])
                return tuple(out)

            accs = lax.fori_loop(0, K, rbody, tuple(bch * G))
            for jj in range(G):
                node = g * G + jj
                for c in range(nch):
                    v = accs[jj * nch + c]
                    if act:
                        v = jnp.maximum(v, 0.01 * v)
                    out_v[node, pl.ds(c * L, L)] = v

        for g0 in range(3):
            gather(g0, rows_bufs[g0], sems[g0]).start()

        def body(i, _):
            for j in range(4):
                g = i * 4 + j
                gather(g, rows_bufs[j], sems[j]).wait()

                @pl.when(g + 3 < NG)
                def _():
                    jn = (j + 3) % 4
                    gather(g + 3, rows_bufs[jn], sems[jn]).start()

                compute(g, rows_bufs[j])
            return 0

        lax.fori_loop(0, NG // 4, body, 0)
        pltpu.sync_copy(out_v, out_hbm.at[pl.ds(wid * CPW, CPW)])

    return _combine


_combine_noact = _make_combine(False)
_combine_act = _make_combine(True)


_MM_BN = 512
_KW = 8


def _mm_body(h_ref, m_ref, o_ref):
    h_blk = h_ref[...]
    for kk in range(K // _KW):
        r = jnp.dot(h_blk, m_ref[:, kk * _KW * D:(kk + 1) * _KW * D],
                    preferred_element_type=_f32)
        for j in range(_KW):
            o_ref[kk * _KW + j, :, :] = r[:, j * D:(j + 1) * D]


def _tc_matmul(h, m):
    bn = _MM_BN
    return pl.pallas_call(
        _mm_body,
        grid=(NP // bn,),
        in_specs=[
            pl.BlockSpec((bn, D), lambda i: (i, 0)),
            pl.BlockSpec((D, K * D), lambda i: (0, 0)),
        ],
        out_specs=pl.BlockSpec((K, bn, D), lambda i: (0, i, 0)),
        out_shape=jax.ShapeDtypeStruct((K, NP, D), _f32),
    )(h, m)


def kernel(x, pos, neighbors, W0, b0, W1, b1, W2, b2):
    n = x.shape[0]
    x_p = jnp.zeros((NP, D), _f32).at[:n].set(x)
    pos_p = jnp.zeros((NP, 3), _f32).at[:n].set(pos)
    nbr_p = jnp.zeros((NP, K), _i32).at[:n].set(neighbors)

    w_flat, fidx_flat = _sc_prep(pos_p[:, 0], pos_p[:, 1], pos_p[:, 2], nbr_p)
    w3 = w_flat.reshape(NW, NG * G * K)
    fidx3 = fidx_flat.reshape(NW, NG, G * K)

    h = x_p
    for wgt, b, act in ((W0, b0, False), (W1, b1, True), (W2, b2, False)):
        m = wgt.reshape(K, D, D).transpose(1, 0, 2).reshape(D, K * D)
        p = _tc_matmul(h, m).reshape(K * NP, D)
        comb = _combine_act if act else _combine_noact
        h = comb(p, fidx3, w3, b)
    return h[:n]
